# Initial kernel scaffold; baseline (speedup 1.0000x reference)
import functools

import jax
import jax.numpy as jnp
from jax import lax
from jax.experimental import pallas as pl
from jax.experimental.pallas import tpu as pltpu
from jax.experimental.pallas import tpu_sc as plsc

N_NODES = 10000
NP = 10240
E = 320000
EB = 128
NBATCH = E // EB
TILES = 16
ROWS_PT = NP // TILES
ZR = 160

_F32 = jnp.float32
_ZERO16 = jnp.zeros((16,), _F32)
_ONE16 = jnp.ones((16,), _F32)


def _fill_zero(ref, rows, cols):
    def body(r, _):
        for j in range(cols // 16):
            ref[r, pl.ds(j * 16, 16)] = _ZERO16
        return 0
    lax.fori_loop(0, rows, body, 0)


def _fill_zero_1d(ref, n):
    def body(r, _):
        ref[pl.ds(r * 16, 16)] = _ZERO16
        return 0
    lax.fori_loop(0, n // 16, body, 0)


def _edge_loop(sid, table_h, src_h, dst_h, acc, src_v, dst_v, rows_v, sem,
               cacc=None, ones_v=None):
    nb = jnp.where(sid < (NBATCH % TILES), NBATCH // TILES + 1,
                   NBATCH // TILES)

    def body(j, _):
        base = (sid + TILES * j) * EB
        pltpu.sync_copy(src_h.at[pl.ds(base, EB)], src_v)
        pltpu.sync_copy(dst_h.at[pl.ds(base, EB)], dst_v)
        pltpu.async_copy(table_h.at[src_v], rows_v, sem).wait()
        pltpu.sync_copy(rows_v, acc.at[dst_v], add=True)
        if cacc is not None:
            pltpu.sync_copy(ones_v, cacc.at[dst_v], add=True)
        return 0

    lax.fori_loop(0, nb, body, 0)


def _make_seg_sum(d2, with_cnt):
    mesh = plsc.VectorSubcoreMesh(core_axis_name="c", subcore_axis_name="s")
    out_type = [jax.ShapeDtypeStruct((NP, 2 * d2), _F32)]
    if with_cnt:
        out_type.append(jax.ShapeDtypeStruct((NP,), _F32))
    scratch = [
        pltpu.VMEM_SHARED((NP, d2), _F32),
        pltpu.VMEM((EB,), jnp.int32),
        pltpu.VMEM((EB,), jnp.int32),
        pltpu.VMEM((EB, d2), _F32),
        pltpu.VMEM((ZR, d2), _F32),
        pltpu.SemaphoreType.DMA,
    ]
    if with_cnt:
        scratch += [
            pltpu.VMEM_SHARED((NP,), _F32),
            pltpu.VMEM((EB,), _F32),
            pltpu.VMEM((ROWS_PT,), _F32),
        ]

    @functools.partial(pl.kernel, mesh=mesh, out_type=out_type,
                       scratch_types=scratch)
    def seg_sum(t0_h, t1_h, src_h, dst_h, *rest):
        if with_cnt:
            (out_h, cnt_h, acc, src_v, dst_v, rows_v, zbuf, sem,
             cacc, ones_v, zrow) = rest
        else:
            out_h, acc, src_v, dst_v, rows_v, zbuf, sem = rest
            cacc = ones_v = zrow = None
        cid = lax.axis_index("c")
        sid = lax.axis_index("s")
        row0 = sid * ROWS_PT

        _fill_zero(zbuf, ZR, d2)
        for z in range(ROWS_PT // ZR):
            pltpu.sync_copy(zbuf, acc.at[pl.ds(row0 + z * ZR, ZR)])
        if with_cnt:
            _fill_zero_1d(zrow, ROWS_PT)
            pltpu.sync_copy(zrow, cacc.at[pl.ds(row0, ROWS_PT)])

            def ones_body(j, _):
                ones_v[pl.ds(j * 16, 16)] = _ONE16
                return 0
            lax.fori_loop(0, EB // 16, ones_body, 0)
        plsc.subcore_barrier()

        @pl.when(cid == 0)
        def _():
            _edge_loop(sid, t0_h, src_h, dst_h, acc, src_v, dst_v, rows_v,
                       sem, cacc, ones_v)

        @pl.when(cid == 1)
        def _():
            _edge_loop(sid, t1_h, src_h, dst_h, acc, src_v, dst_v, rows_v,
                       sem, cacc, ones_v)

        plsc.subcore_barrier()

        @pl.when(cid == 0)
        def _():
            pltpu.sync_copy(acc.at[pl.ds(row0, ROWS_PT)],
                            out_h.at[pl.ds(row0, ROWS_PT), pl.ds(0, d2)])
            if with_cnt:
                pltpu.sync_copy(cacc.at[pl.ds(row0, ROWS_PT)],
                                cnt_h.at[pl.ds(row0, ROWS_PT)])

        @pl.when(cid == 1)
        def _():
            pltpu.sync_copy(acc.at[pl.ds(row0, ROWS_PT)],
                            out_h.at[pl.ds(row0, ROWS_PT), pl.ds(d2, d2)])

    return seg_sum


_seg_sum_64 = _make_seg_sum(64, with_cnt=True)
_seg_sum_128 = _make_seg_sum(128, with_cnt=False)


def _make_seg_mean16():
    mesh = plsc.VectorSubcoreMesh(core_axis_name="c", subcore_axis_name="s")
    out_type = jax.ShapeDtypeStruct((NP, 16), _F32)
    scratch = [
        pltpu.VMEM_SHARED((NP, 16), _F32),
        pltpu.VMEM((EB,), jnp.int32),
        pltpu.VMEM((EB,), jnp.int32),
        pltpu.VMEM((EB, 16), _F32),
        pltpu.VMEM((ROWS_PT, 16), _F32),
        pltpu.VMEM((ROWS_PT, 16), _F32),
        pltpu.VMEM((ROWS_PT,), _F32),
        pltpu.SemaphoreType.DMA,
    ]

    @functools.partial(pl.kernel, mesh=mesh, out_type=out_type,
                       scratch_types=scratch)
    def seg_mean16(p_h, init_h, src_h, dst_h, cnt_h, out_h,
                   acc, src_v, dst_v, rows_v, accv, outv, cntv, sem):
        cid = lax.axis_index("c")
        sid = lax.axis_index("s")
        row0 = sid * ROWS_PT

        @pl.when(cid == 0)
        def _():
            pltpu.sync_copy(init_h.at[pl.ds(row0, ROWS_PT)],
                            acc.at[pl.ds(row0, ROWS_PT)])
            plsc.subcore_barrier()
            _edge_loop(sid, p_h, src_h, dst_h, acc, src_v, dst_v, rows_v,
                       sem)
            plsc.subcore_barrier()
            pltpu.sync_copy(acc.at[pl.ds(row0, ROWS_PT)], accv)
            pltpu.sync_copy(cnt_h.at[pl.ds(row0, ROWS_PT)], cntv)
            iota16 = lax.iota(jnp.int32, 16)

            def row_body(r, _):
                ridx = jnp.full((16,), r, jnp.int32)
                cval = plsc.load_gather(cntv, [ridx])
                inv = 1.0 / jnp.maximum(cval, 1.0)
                row = plsc.load_gather(accv, [ridx, iota16])
                plsc.store_scatter(outv, [ridx, iota16], row * inv)
                return 0

            lax.fori_loop(0, ROWS_PT, row_body, 0)
            pltpu.sync_copy(outv, out_h.at[pl.ds(row0, ROWS_PT)])

    return seg_mean16


_seg_mean16 = _make_seg_mean16()

BM = 256
GRID = NP // BM


def _l1_body(cnt_ref, agg_ref, x_ref, wl_ref, wr_ref, b_ref,
             h1a_ref, h1b_ref):
    cnt = cnt_ref[...][:, 0:1]
    mean = agg_ref[...] * (1.0 / jnp.maximum(cnt, 1.0))
    h = jnp.dot(mean, wl_ref[...], preferred_element_type=_F32)
    h = h + jnp.dot(x_ref[...], wr_ref[...], preferred_element_type=_F32)
    h = jnp.maximum(h + b_ref[...][0:1, :], 0.0)
    h1a_ref[...] = h[:, :128]
    h1b_ref[...] = h[:, 128:]


def _tc_layer1(cntb, agg1, xp, Wl1, Wr1, b1r):
    return pl.pallas_call(
        _l1_body,
        grid=(GRID,),
        in_specs=[
            pl.BlockSpec((BM, 128), lambda i: (i, 0)),
            pl.BlockSpec((BM, 128), lambda i: (i, 0)),
            pl.BlockSpec((BM, 128), lambda i: (i, 0)),
            pl.BlockSpec((128, 256), lambda i: (0, 0)),
            pl.BlockSpec((128, 256), lambda i: (0, 0)),
            pl.BlockSpec((8, 256), lambda i: (0, 0)),
        ],
        out_specs=[
            pl.BlockSpec((BM, 128), lambda i: (i, 0)),
            pl.BlockSpec((BM, 128), lambda i: (i, 0)),
        ],
        out_shape=[
            jax.ShapeDtypeStruct((NP, 128), _F32),
            jax.ShapeDtypeStruct((NP, 128), _F32),
        ],
    )(cntb, agg1, xp, Wl1, Wr1, b1r)


def _l2_body(cnt_ref, agg_ref, h1a_ref, h1b_ref, wl2_ref, wr2a_ref,
             wr2b_ref, b2_ref, wl3_ref, wr3_ref, b3_ref, p_ref, init_ref):
    cnt = cnt_ref[...][:, 0:1]
    cmax = jnp.maximum(cnt, 1.0)
    mean = agg_ref[...] * (1.0 / cmax)
    h = jnp.dot(mean, wl2_ref[...], preferred_element_type=_F32)
    h = h + jnp.dot(h1a_ref[...], wr2a_ref[...], preferred_element_type=_F32)
    h = h + jnp.dot(h1b_ref[...], wr2b_ref[...], preferred_element_type=_F32)
    h = jnp.maximum(h + b2_ref[...][0:1, :], 0.0)
    p_ref[...] = jnp.dot(h, wl3_ref[...], preferred_element_type=_F32)
    r = jnp.dot(h, wr3_ref[...], preferred_element_type=_F32)
    init_ref[...] = (r + b3_ref[...][0:1, :]) * cmax


def _tc_layer23(cntb, agg2, h1a, h1b, Wl2, Wr2a, Wr2b, b2r, Wl3p, Wr3p, b3r):
    return pl.pallas_call(
        _l2_body,
        grid=(GRID,),
        in_specs=[
            pl.BlockSpec((BM, 128), lambda i: (i, 0)),
            pl.BlockSpec((BM, 256), lambda i: (i, 0)),
            pl.BlockSpec((BM, 128), lambda i: (i, 0)),
            pl.BlockSpec((BM, 128), lambda i: (i, 0)),
            pl.BlockSpec((256, 256), lambda i: (0, 0)),
            pl.BlockSpec((128, 256), lambda i: (0, 0)),
            pl.BlockSpec((128, 256), lambda i: (0, 0)),
            pl.BlockSpec((8, 256), lambda i: (0, 0)),
            pl.BlockSpec((256, 128), lambda i: (0, 0)),
            pl.BlockSpec((256, 128), lambda i: (0, 0)),
            pl.BlockSpec((8, 128), lambda i: (0, 0)),
        ],
        out_specs=[
            pl.BlockSpec((BM, 128), lambda i: (i, 0)),
            pl.BlockSpec((BM, 128), lambda i: (i, 0)),
        ],
        out_shape=[
            jax.ShapeDtypeStruct((NP, 128), _F32),
            jax.ShapeDtypeStruct((NP, 128), _F32),
        ],
    )(cntb, agg2, h1a, h1b, Wl2, Wr2a, Wr2b, b2r, Wl3p, Wr3p, b3r)


def kernel(x, edge_index, Wl1, Wr1, b1, Wl2, Wr2, b2, Wl3, Wr3, b3):
    src = edge_index[0].astype(jnp.int32)
    dst = edge_index[1].astype(jnp.int32)

    xp = jnp.pad(x, ((0, NP - N_NODES), (0, 0)))
    x0 = xp[:, :64]
    x1 = xp[:, 64:]

    agg1, cnt = _seg_sum_64(x0, x1, src, dst)
    cntb = jnp.broadcast_to(cnt[:, None], (NP, 128))

    b1r = jnp.broadcast_to(b1[None, :], (8, 256))
    h1a, h1b = _tc_layer1(cntb, agg1, xp, Wl1, Wr1, b1r)

    agg2 = _seg_sum_128(h1a, h1b, src, dst)

    b2r = jnp.broadcast_to(b2[None, :], (8, 256))
    Wl3p = jnp.pad(Wl3, ((0, 0), (0, 126)))
    Wr3p = jnp.pad(Wr3, ((0, 0), (0, 126)))
    b3r = jnp.broadcast_to(jnp.pad(b3, (0, 126))[None, :], (8, 128))
    pout, iout = _tc_layer23(cntb, agg2, h1a, h1b, Wl2, Wr2[:128],
                             Wr2[128:], b2r, Wl3p, Wr3p, b3r)

    p16 = pout[:, :16]
    init16 = iout[:, :16]
    outp = _seg_mean16(p16, init16, src, dst, cnt)
    return outp[:N_NODES, :2]


# speedup vs baseline: 10.5039x; 10.5039x over previous
import functools

import jax
import jax.numpy as jnp
from jax import lax
from jax.experimental import pallas as pl
from jax.experimental.pallas import tpu as pltpu
from jax.experimental.pallas import tpu_sc as plsc

N_NODES = 10000
NP = 10240
E = 320000
EB = 128
NBATCH = E // EB
TILES = 16
ROWS_PT = NP // TILES
ZR = 160

_F32 = jnp.float32


def _fill_zero(ref, rows, cols):
    zero16 = jnp.zeros((16,), _F32)

    def body(r, _):
        for j in range(cols // 16):
            ref[r, pl.ds(j * 16, 16)] = zero16
        return 0
    lax.fori_loop(0, rows, body, 0)


def _fill_zero_1d(ref, n):
    zero16 = jnp.zeros((16,), _F32)

    def body(r, _):
        ref[pl.ds(r * 16, 16)] = zero16
        return 0
    lax.fori_loop(0, n // 16, body, 0)


def _memset_acc_rows(zsrc, acc, row0, zrows):
    off = 0
    while off < ROWS_PT:
        n = min(zrows, ROWS_PT - off)
        pltpu.sync_copy(zsrc.at[pl.ds(0, n)],
                        acc.at[pl.ds(row0 + off, n)])
        off += n


def _edge_loop(sid, table_h, src_h, dst_h, acc, srcv, dstv, rows_v,
               semi, semg, sems, k, k0b, nbatch, cacc=None, ones_v=None):
    nfull = nbatch // TILES
    rem = nbatch % TILES
    assert nfull % k == 0
    nsup = nfull // k
    o_s = k0b + sid * nfull + jnp.minimum(sid, rem)

    def fire_gather(b0, i):
        p = i % 2
        return pltpu.async_copy(
            table_h.at[srcv.at[pl.ds(i * EB, EB)]],
            rows_v.at[pl.ds(p * EB, EB)], semg)

    def fire_scatter(i):
        p = i % 2
        cps = [pltpu.async_copy(rows_v.at[pl.ds(p * EB, EB)],
                                acc.at[dstv.at[i]], sems, add=True)]
        if cacc is not None:
            cps.append(pltpu.async_copy(ones_v, cacc.at[dstv.at[i]], sems,
                                        add=True))
        return cps

    def do_group(b0, kk):
        e0 = b0 * EB
        cps = [pltpu.async_copy(src_h.at[pl.ds(e0, kk * EB)],
                                srcv.at[pl.ds(0, kk * EB)], semi)]
        cps += [pltpu.async_copy(dst_h.at[pl.ds(e0 + i * EB, EB)],
                                 dstv.at[i], semi)
                for i in range(kk)]
        for cp in cps:
            cp.wait()
        g = fire_gather(b0, 0)
        sc_prev = []
        for i in range(kk):
            g.wait()
            for cp in sc_prev:
                cp.wait()
            if i + 1 < kk:
                g = fire_gather(b0, i + 1)
            sc_prev = fire_scatter(i)
        for cp in sc_prev:
            cp.wait()

    def body(g, _):
        do_group(o_s + g * k, k)
        return 0

    lax.fori_loop(0, nsup, body, 0)

    @pl.when(sid < rem)
    def _():
        do_group(o_s + nfull, 1)


def _make_seg_sum_l1():
    mesh = plsc.VectorSubcoreMesh(core_axis_name="c", subcore_axis_name="s")
    out_type = [
        jax.ShapeDtypeStruct((NP, 128), _F32),
        jax.ShapeDtypeStruct((NP, 128), _F32),
        jax.ShapeDtypeStruct((NP,), _F32),
        jax.ShapeDtypeStruct((NP,), _F32),
    ]
    k = 6
    scratch = [
        pltpu.VMEM_SHARED((NP, 128), _F32),
        pltpu.VMEM((k * EB,), jnp.int32),
        pltpu.VMEM((k, EB), jnp.int32),
        pltpu.VMEM((2 * EB, 128), _F32),
        pltpu.SemaphoreType.DMA,
        pltpu.SemaphoreType.DMA,
        pltpu.SemaphoreType.DMA,
        pltpu.VMEM_SHARED((NP,), _F32),
        pltpu.VMEM((EB,), _F32),
        pltpu.VMEM((ROWS_PT,), _F32),
    ]

    @functools.partial(pl.kernel, mesh=mesh, out_type=out_type,
                       scratch_types=scratch)
    def seg_sum_l1(x_h, src_h, dst_h, p0_h, p1_h, c0_h, c1_h,
                   acc, srcv, dstv, rows_v, semi, semg, sems,
                   cacc, ones_v, zrow):
        cid = lax.axis_index("c")
        sid = lax.axis_index("s")
        row0 = sid * ROWS_PT

        _fill_zero(rows_v, 2 * EB, 128)
        _memset_acc_rows(rows_v, acc, row0, 2 * EB)
        _fill_zero_1d(zrow, ROWS_PT)
        pltpu.sync_copy(zrow, cacc.at[pl.ds(row0, ROWS_PT)])

        one16 = jnp.ones((16,), _F32)

        def ones_body(j, _):
            ones_v[pl.ds(j * 16, 16)] = one16
            return 0
        lax.fori_loop(0, EB // 16, ones_body, 0)
        plsc.subcore_barrier()

        half = NBATCH // 2

        @pl.when(cid == 0)
        def _():
            _edge_loop(sid, x_h, src_h, dst_h, acc, srcv, dstv, rows_v,
                       semi, semg, sems, k, 0, half, cacc, ones_v)

        @pl.when(cid == 1)
        def _():
            _edge_loop(sid, x_h, src_h, dst_h, acc, srcv, dstv, rows_v,
                       semi, semg, sems, k, half, half, cacc, ones_v)

        plsc.subcore_barrier()

        @pl.when(cid == 0)
        def _():
            pltpu.sync_copy(acc.at[pl.ds(row0, ROWS_PT)],
                            p0_h.at[pl.ds(row0, ROWS_PT)])
            pltpu.sync_copy(cacc.at[pl.ds(row0, ROWS_PT)],
                            c0_h.at[pl.ds(row0, ROWS_PT)])

        @pl.when(cid == 1)
        def _():
            pltpu.sync_copy(acc.at[pl.ds(row0, ROWS_PT)],
                            p1_h.at[pl.ds(row0, ROWS_PT)])
            pltpu.sync_copy(cacc.at[pl.ds(row0, ROWS_PT)],
                            c1_h.at[pl.ds(row0, ROWS_PT)])

    return seg_sum_l1


def _make_seg_sum_fsplit():
    mesh = plsc.VectorSubcoreMesh(core_axis_name="c", subcore_axis_name="s")
    out_type = [jax.ShapeDtypeStruct((NP, 256), _F32)]
    k = 12
    scratch = [
        pltpu.VMEM_SHARED((NP, 128), _F32),
        pltpu.VMEM((k * EB,), jnp.int32),
        pltpu.VMEM((k, EB), jnp.int32),
        pltpu.VMEM((2 * EB, 128), _F32),
        pltpu.SemaphoreType.DMA,
        pltpu.SemaphoreType.DMA,
        pltpu.SemaphoreType.DMA,
    ]

    @functools.partial(pl.kernel, mesh=mesh, out_type=out_type,
                       scratch_types=scratch)
    def seg_sum_fs(t0_h, t1_h, src_h, dst_h, out_h,
                   acc, srcv, dstv, rows_v, semi, semg, sems):
        cid = lax.axis_index("c")
        sid = lax.axis_index("s")
        row0 = sid * ROWS_PT

        _fill_zero(rows_v, 2 * EB, 128)
        _memset_acc_rows(rows_v, acc, row0, 2 * EB)
        plsc.subcore_barrier()

        @pl.when(cid == 0)
        def _():
            _edge_loop(sid, t0_h, src_h, dst_h, acc, srcv, dstv, rows_v,
                       semi, semg, sems, k, 0, NBATCH)

        @pl.when(cid == 1)
        def _():
            _edge_loop(sid, t1_h, src_h, dst_h, acc, srcv, dstv, rows_v,
                       semi, semg, sems, k, 0, NBATCH)

        plsc.subcore_barrier()

        @pl.when(cid == 0)
        def _():
            pltpu.sync_copy(acc.at[pl.ds(row0, ROWS_PT)],
                            out_h.at[pl.ds(row0, ROWS_PT), pl.ds(0, 128)])

        @pl.when(cid == 1)
        def _():
            pltpu.sync_copy(acc.at[pl.ds(row0, ROWS_PT)],
                            out_h.at[pl.ds(row0, ROWS_PT), pl.ds(128, 128)])

    return seg_sum_fs


_seg_sum_l1 = _make_seg_sum_l1()
_seg_sum_128 = _make_seg_sum_fsplit()


def _make_seg_mean16():
    mesh = plsc.VectorSubcoreMesh(core_axis_name="c", subcore_axis_name="s")
    out_type = [
        jax.ShapeDtypeStruct((NP, 16), _F32),
        jax.ShapeDtypeStruct((NP, 16), _F32),
    ]
    k = 6
    scratch = [
        pltpu.VMEM_SHARED((NP, 16), _F32),
        pltpu.VMEM((k * EB,), jnp.int32),
        pltpu.VMEM((k, EB), jnp.int32),
        pltpu.VMEM((2 * EB, 16), _F32),
        pltpu.SemaphoreType.DMA,
        pltpu.SemaphoreType.DMA,
        pltpu.SemaphoreType.DMA,
    ]

    @functools.partial(
        pl.kernel, mesh=mesh, out_type=out_type, scratch_types=scratch,
        compiler_params=pltpu.CompilerParams(use_tc_tiling_on_sc=False))
    def seg_mean16(p_h, init_h, src_h, dst_h, o0_h, o1_h,
                   acc, srcv, dstv, rows_v, semi, semg, sems):
        cid = lax.axis_index("c")
        sid = lax.axis_index("s")
        row0 = sid * ROWS_PT
        half = NBATCH // 2

        @pl.when(cid == 0)
        def _():
            pltpu.sync_copy(init_h.at[pl.ds(row0, ROWS_PT)],
                            acc.at[pl.ds(row0, ROWS_PT)])
            plsc.subcore_barrier()
            _edge_loop(sid, p_h, src_h, dst_h, acc, srcv, dstv, rows_v,
                       semi, semg, sems, k, 0, half)
            plsc.subcore_barrier()
            pltpu.sync_copy(acc.at[pl.ds(row0, ROWS_PT)],
                            o0_h.at[pl.ds(row0, ROWS_PT)])

        @pl.when(cid == 1)
        def _():
            _fill_zero(rows_v, 2 * EB, 16)
            _memset_acc_rows(rows_v, acc, row0, 2 * EB)
            plsc.subcore_barrier()
            _edge_loop(sid, p_h, src_h, dst_h, acc, srcv, dstv, rows_v,
                       semi, semg, sems, k, half, half)
            plsc.subcore_barrier()
            pltpu.sync_copy(acc.at[pl.ds(row0, ROWS_PT)],
                            o1_h.at[pl.ds(row0, ROWS_PT)])

    return seg_mean16


_seg_mean16 = _make_seg_mean16()

BM = 256
GRID = NP // BM


def _l1_body(cnt_ref, p0_ref, p1_ref, x_ref, wl_ref, wr_ref, b_ref,
             h1a_ref, h1b_ref):
    cnt = cnt_ref[...][:, 0:1]
    agg = p0_ref[...] + p1_ref[...]
    mean = agg * (1.0 / jnp.maximum(cnt, 1.0))
    h = jnp.dot(mean, wl_ref[...], preferred_element_type=_F32)
    h = h + jnp.dot(x_ref[...], wr_ref[...], preferred_element_type=_F32)
    h = jnp.maximum(h + b_ref[...][0:1, :], 0.0)
    h1a_ref[...] = h[:, :128]
    h1b_ref[...] = h[:, 128:]


def _tc_layer1(cntb, p0, p1, xp, Wl1, Wr1, b1r):
    return pl.pallas_call(
        _l1_body,
        grid=(GRID,),
        in_specs=[
            pl.BlockSpec((BM, 128), lambda i: (i, 0)),
            pl.BlockSpec((BM, 128), lambda i: (i, 0)),
            pl.BlockSpec((BM, 128), lambda i: (i, 0)),
            pl.BlockSpec((BM, 128), lambda i: (i, 0)),
            pl.BlockSpec((128, 256), lambda i: (0, 0)),
            pl.BlockSpec((128, 256), lambda i: (0, 0)),
            pl.BlockSpec((8, 256), lambda i: (0, 0)),
        ],
        out_specs=[
            pl.BlockSpec((BM, 128), lambda i: (i, 0)),
            pl.BlockSpec((BM, 128), lambda i: (i, 0)),
        ],
        out_shape=[
            jax.ShapeDtypeStruct((NP, 128), _F32),
            jax.ShapeDtypeStruct((NP, 128), _F32),
        ],
    )(cntb, p0, p1, xp, Wl1, Wr1, b1r)


def _l2_body(cnt_ref, agg_ref, h1a_ref, h1b_ref, wl2_ref, wr2a_ref,
             wr2b_ref, b2_ref, wl3_ref, wr3_ref, b3_ref, p_ref, init_ref):
    cnt = cnt_ref[...][:, 0:1]
    cmax = jnp.maximum(cnt, 1.0)
    mean = agg_ref[...] * (1.0 / cmax)
    h = jnp.dot(mean, wl2_ref[...], preferred_element_type=_F32)
    h = h + jnp.dot(h1a_ref[...], wr2a_ref[...], preferred_element_type=_F32)
    h = h + jnp.dot(h1b_ref[...], wr2b_ref[...], preferred_element_type=_F32)
    h = jnp.maximum(h + b2_ref[...][0:1, :], 0.0)
    p_ref[...] = jnp.dot(h, wl3_ref[...], preferred_element_type=_F32)
    r = jnp.dot(h, wr3_ref[...], preferred_element_type=_F32)
    init_ref[...] = (r + b3_ref[...][0:1, :]) * cmax


def _tc_layer23(cntb, agg2, h1a, h1b, Wl2, Wr2a, Wr2b, b2r, Wl3p, Wr3p, b3r):
    return pl.pallas_call(
        _l2_body,
        grid=(GRID,),
        in_specs=[
            pl.BlockSpec((BM, 128), lambda i: (i, 0)),
            pl.BlockSpec((BM, 256), lambda i: (i, 0)),
            pl.BlockSpec((BM, 128), lambda i: (i, 0)),
            pl.BlockSpec((BM, 128), lambda i: (i, 0)),
            pl.BlockSpec((256, 256), lambda i: (0, 0)),
            pl.BlockSpec((128, 256), lambda i: (0, 0)),
            pl.BlockSpec((128, 256), lambda i: (0, 0)),
            pl.BlockSpec((8, 256), lambda i: (0, 0)),
            pl.BlockSpec((256, 128), lambda i: (0, 0)),
            pl.BlockSpec((256, 128), lambda i: (0, 0)),
            pl.BlockSpec((8, 128), lambda i: (0, 0)),
        ],
        out_specs=[
            pl.BlockSpec((BM, 128), lambda i: (i, 0)),
            pl.BlockSpec((BM, 128), lambda i: (i, 0)),
        ],
        out_shape=[
            jax.ShapeDtypeStruct((NP, 128), _F32),
            jax.ShapeDtypeStruct((NP, 128), _F32),
        ],
    )(cntb, agg2, h1a, h1b, Wl2, Wr2a, Wr2b, b2r, Wl3p, Wr3p, b3r)


def _finish_body(cnt_ref, a0_ref, a1_ref, out_ref):
    cnt = cnt_ref[...][:, 0:1]
    out_ref[...] = ((a0_ref[...] + a1_ref[...])
                    * (1.0 / jnp.maximum(cnt, 1.0)))


def _tc_finish(cntb, a0, a1):
    return pl.pallas_call(
        _finish_body,
        grid=(GRID,),
        in_specs=[
            pl.BlockSpec((BM, 128), lambda i: (i, 0)),
            pl.BlockSpec((BM, 16), lambda i: (i, 0)),
            pl.BlockSpec((BM, 16), lambda i: (i, 0)),
        ],
        out_specs=pl.BlockSpec((BM, 16), lambda i: (i, 0)),
        out_shape=jax.ShapeDtypeStruct((NP, 16), _F32),
    )(cntb, a0, a1)


def kernel(x, edge_index, Wl1, Wr1, b1, Wl2, Wr2, b2, Wl3, Wr3, b3):
    src = edge_index[0].astype(jnp.int32)
    dst = edge_index[1].astype(jnp.int32)

    xp = jnp.pad(x, ((0, NP - N_NODES), (0, 0)))

    p0, p1, c0, c1 = _seg_sum_l1(xp, src, dst)
    cnt = c0 + c1
    cntb = jnp.broadcast_to(cnt[:, None], (NP, 128))

    b1r = jnp.broadcast_to(b1[None, :], (8, 256))
    h1a, h1b = _tc_layer1(cntb, p0, p1, xp, Wl1, Wr1, b1r)

    agg2, = _seg_sum_128(h1a, h1b, src, dst)

    b2r = jnp.broadcast_to(b2[None, :], (8, 256))
    Wl3p = jnp.pad(Wl3, ((0, 0), (0, 126)))
    Wr3p = jnp.pad(Wr3, ((0, 0), (0, 126)))
    b3r = jnp.broadcast_to(jnp.pad(b3, (0, 126))[None, :], (8, 128))
    pout, iout = _tc_layer23(cntb, agg2, h1a, h1b, Wl2, Wr2[:128],
                             Wr2[128:], b2r, Wl3p, Wr3p, b3r)

    p16 = pout[:, :16]
    init16 = iout[:, :16]
    a0, a1 = _seg_mean16(p16, init16, src, dst)
    outp = _tc_finish(cntb, a0, a1)
    return outp[:N_NODES, :2]


# speedup vs baseline: 10.6392x; 1.0129x over previous
import functools

import jax
import jax.numpy as jnp
from jax import lax
from jax.experimental import pallas as pl
from jax.experimental.pallas import tpu as pltpu
from jax.experimental.pallas import tpu_sc as plsc

N_NODES = 10000
NP = 10240
E = 320000
EB = 128
NBATCH = E // EB
TILES = 16
ROWS_PT = NP // TILES
ZR = 160

_F32 = jnp.float32


def _fill_zero(ref, rows, cols):
    zero16 = jnp.zeros((16,), _F32)

    def body(r, _):
        for j in range(cols // 16):
            ref[r, pl.ds(j * 16, 16)] = zero16
        return 0
    lax.fori_loop(0, rows, body, 0)


def _fill_zero_1d(ref, n):
    zero16 = jnp.zeros((16,), _F32)

    def body(r, _):
        ref[pl.ds(r * 16, 16)] = zero16
        return 0
    lax.fori_loop(0, n // 16, body, 0)


def _memset_acc_rows(zsrc, acc, row0, zrows):
    off = 0
    while off < ROWS_PT:
        n = min(zrows, ROWS_PT - off)
        pltpu.sync_copy(zsrc.at[pl.ds(0, n)],
                        acc.at[pl.ds(row0 + off, n)])
        off += n


def _edge_loop(sid, table_h, src_h, dst_h, acc, srcv, dstv, rows_v,
               semi, semg, sems, k, k0b, nbatch, cacc=None, ones_v=None):
    nfull = nbatch // TILES
    rem = nbatch % TILES
    assert nfull % k == 0
    nsup = nfull // k
    o_s = k0b + sid * nfull + jnp.minimum(sid, rem)

    def fire_gather(b0, i):
        p = i % 2
        return pltpu.async_copy(
            table_h.at[srcv.at[pl.ds(i * EB, EB)]],
            rows_v.at[pl.ds(p * EB, EB)], semg)

    def fire_scatter(i):
        p = i % 2
        cps = [pltpu.async_copy(rows_v.at[pl.ds(p * EB, EB)],
                                acc.at[dstv.at[i]], sems, add=True)]
        if cacc is not None:
            cps.append(pltpu.async_copy(ones_v, cacc.at[dstv.at[i]], sems,
                                        add=True))
        return cps

    def do_group(b0, kk):
        e0 = b0 * EB
        cps = [pltpu.async_copy(src_h.at[pl.ds(e0, kk * EB)],
                                srcv.at[pl.ds(0, kk * EB)], semi)]
        cps += [pltpu.async_copy(dst_h.at[pl.ds(e0 + i * EB, EB)],
                                 dstv.at[i], semi)
                for i in range(kk)]
        for cp in cps:
            cp.wait()
        g = fire_gather(b0, 0)
        sc_prev = []
        for i in range(kk):
            g.wait()
            for cp in sc_prev:
                cp.wait()
            if i + 1 < kk:
                g = fire_gather(b0, i + 1)
            sc_prev = fire_scatter(i)
        for cp in sc_prev:
            cp.wait()

    def body(g, _):
        do_group(o_s + g * k, k)
        return 0

    lax.fori_loop(0, nsup, body, 0)

    @pl.when(sid < rem)
    def _():
        do_group(o_s + nfull, 1)


def _make_seg_sum_l1():
    mesh = plsc.VectorSubcoreMesh(core_axis_name="c", subcore_axis_name="s")
    out_type = [
        jax.ShapeDtypeStruct((NP, 128), _F32),
        jax.ShapeDtypeStruct((NP, 128), _F32),
        jax.ShapeDtypeStruct((NP,), _F32),
        jax.ShapeDtypeStruct((NP,), _F32),
    ]
    k = 13
    scratch = [
        pltpu.VMEM_SHARED((NP, 128), _F32),
        pltpu.VMEM((k * EB,), jnp.int32),
        pltpu.VMEM((k, EB), jnp.int32),
        pltpu.VMEM((2 * EB, 128), _F32),
        pltpu.SemaphoreType.DMA,
        pltpu.SemaphoreType.DMA,
        pltpu.SemaphoreType.DMA,
        pltpu.VMEM_SHARED((NP,), _F32),
        pltpu.VMEM((EB,), _F32),
        pltpu.VMEM((ROWS_PT,), _F32),
    ]

    @functools.partial(pl.kernel, mesh=mesh, out_type=out_type,
                       scratch_types=scratch)
    def seg_sum_l1(x_h, src_h, dst_h, p0_h, p1_h, c0_h, c1_h,
                   acc, srcv, dstv, rows_v, semi, semg, sems,
                   cacc, ones_v, zrow):
        cid = lax.axis_index("c")
        sid = lax.axis_index("s")
        row0 = sid * ROWS_PT

        _fill_zero(rows_v, 2 * EB, 128)
        _memset_acc_rows(rows_v, acc, row0, 2 * EB)
        _fill_zero_1d(zrow, ROWS_PT)
        pltpu.sync_copy(zrow, cacc.at[pl.ds(row0, ROWS_PT)])

        one16 = jnp.ones((16,), _F32)

        def ones_body(j, _):
            ones_v[pl.ds(j * 16, 16)] = one16
            return 0
        lax.fori_loop(0, EB // 16, ones_body, 0)
        plsc.subcore_barrier()

        half = NBATCH // 2

        @pl.when(cid == 0)
        def _():
            _edge_loop(sid, x_h, src_h, dst_h, acc, srcv, dstv, rows_v,
                       semi, semg, sems, k, 0, half, cacc, ones_v)

        @pl.when(cid == 1)
        def _():
            _edge_loop(sid, x_h, src_h, dst_h, acc, srcv, dstv, rows_v,
                       semi, semg, sems, k, half, half, cacc, ones_v)

        plsc.subcore_barrier()

        @pl.when(cid == 0)
        def _():
            pltpu.sync_copy(acc.at[pl.ds(row0, ROWS_PT)],
                            p0_h.at[pl.ds(row0, ROWS_PT)])
            pltpu.sync_copy(cacc.at[pl.ds(row0, ROWS_PT)],
                            c0_h.at[pl.ds(row0, ROWS_PT)])

        @pl.when(cid == 1)
        def _():
            pltpu.sync_copy(acc.at[pl.ds(row0, ROWS_PT)],
                            p1_h.at[pl.ds(row0, ROWS_PT)])
            pltpu.sync_copy(cacc.at[pl.ds(row0, ROWS_PT)],
                            c1_h.at[pl.ds(row0, ROWS_PT)])

    return seg_sum_l1


def _make_seg_sum_fsplit():
    mesh = plsc.VectorSubcoreMesh(core_axis_name="c", subcore_axis_name="s")
    out_type = [jax.ShapeDtypeStruct((NP, 256), _F32)]
    k = 12
    scratch = [
        pltpu.VMEM_SHARED((NP, 128), _F32),
        pltpu.VMEM((k * EB,), jnp.int32),
        pltpu.VMEM((k, EB), jnp.int32),
        pltpu.VMEM((2 * EB, 128), _F32),
        pltpu.SemaphoreType.DMA,
        pltpu.SemaphoreType.DMA,
        pltpu.SemaphoreType.DMA,
    ]

    @functools.partial(pl.kernel, mesh=mesh, out_type=out_type,
                       scratch_types=scratch)
    def seg_sum_fs(t0_h, t1_h, src_h, dst_h, out_h,
                   acc, srcv, dstv, rows_v, semi, semg, sems):
        cid = lax.axis_index("c")
        sid = lax.axis_index("s")
        row0 = sid * ROWS_PT

        _fill_zero(rows_v, 2 * EB, 128)
        _memset_acc_rows(rows_v, acc, row0, 2 * EB)
        plsc.subcore_barrier()

        @pl.when(cid == 0)
        def _():
            _edge_loop(sid, t0_h, src_h, dst_h, acc, srcv, dstv, rows_v,
                       semi, semg, sems, k, 0, NBATCH)

        @pl.when(cid == 1)
        def _():
            _edge_loop(sid, t1_h, src_h, dst_h, acc, srcv, dstv, rows_v,
                       semi, semg, sems, k, 0, NBATCH)

        plsc.subcore_barrier()

        @pl.when(cid == 0)
        def _():
            pltpu.sync_copy(acc.at[pl.ds(row0, ROWS_PT)],
                            out_h.at[pl.ds(row0, ROWS_PT), pl.ds(0, 128)])

        @pl.when(cid == 1)
        def _():
            pltpu.sync_copy(acc.at[pl.ds(row0, ROWS_PT)],
                            out_h.at[pl.ds(row0, ROWS_PT), pl.ds(128, 128)])

    return seg_sum_fs


_seg_sum_l1 = _make_seg_sum_l1()
_seg_sum_128 = _make_seg_sum_fsplit()


def _make_seg_mean16():
    mesh = plsc.VectorSubcoreMesh(core_axis_name="c", subcore_axis_name="s")
    out_type = [
        jax.ShapeDtypeStruct((NP, 16), _F32),
        jax.ShapeDtypeStruct((NP, 16), _F32),
    ]
    k = 13
    scratch = [
        pltpu.VMEM_SHARED((NP, 16), _F32),
        pltpu.VMEM((k * EB,), jnp.int32),
        pltpu.VMEM((k, EB), jnp.int32),
        pltpu.VMEM((2 * EB, 16), _F32),
        pltpu.SemaphoreType.DMA,
        pltpu.SemaphoreType.DMA,
        pltpu.SemaphoreType.DMA,
    ]

    @functools.partial(
        pl.kernel, mesh=mesh, out_type=out_type, scratch_types=scratch,
        compiler_params=pltpu.CompilerParams(use_tc_tiling_on_sc=False))
    def seg_mean16(p_h, init_h, src_h, dst_h, o0_h, o1_h,
                   acc, srcv, dstv, rows_v, semi, semg, sems):
        cid = lax.axis_index("c")
        sid = lax.axis_index("s")
        row0 = sid * ROWS_PT
        half = NBATCH // 2

        @pl.when(cid == 0)
        def _():
            pltpu.sync_copy(init_h.at[pl.ds(row0, ROWS_PT)],
                            acc.at[pl.ds(row0, ROWS_PT)])
            plsc.subcore_barrier()
            _edge_loop(sid, p_h, src_h, dst_h, acc, srcv, dstv, rows_v,
                       semi, semg, sems, k, 0, half)
            plsc.subcore_barrier()
            pltpu.sync_copy(acc.at[pl.ds(row0, ROWS_PT)],
                            o0_h.at[pl.ds(row0, ROWS_PT)])

        @pl.when(cid == 1)
        def _():
            _fill_zero(rows_v, 2 * EB, 16)
            _memset_acc_rows(rows_v, acc, row0, 2 * EB)
            plsc.subcore_barrier()
            _edge_loop(sid, p_h, src_h, dst_h, acc, srcv, dstv, rows_v,
                       semi, semg, sems, k, half, half)
            plsc.subcore_barrier()
            pltpu.sync_copy(acc.at[pl.ds(row0, ROWS_PT)],
                            o1_h.at[pl.ds(row0, ROWS_PT)])

    return seg_mean16


_seg_mean16 = _make_seg_mean16()

BM = 256
GRID = NP // BM


def _l1_body(cnt_ref, p0_ref, p1_ref, x_ref, wl_ref, wr_ref, b_ref,
             h1a_ref, h1b_ref):
    cnt = cnt_ref[...][:, 0:1]
    agg = p0_ref[...] + p1_ref[...]
    mean = agg * (1.0 / jnp.maximum(cnt, 1.0))
    h = jnp.dot(mean, wl_ref[...], preferred_element_type=_F32)
    h = h + jnp.dot(x_ref[...], wr_ref[...], preferred_element_type=_F32)
    h = jnp.maximum(h + b_ref[...][0:1, :], 0.0)
    h1a_ref[...] = h[:, :128]
    h1b_ref[...] = h[:, 128:]


def _tc_layer1(cntb, p0, p1, xp, Wl1, Wr1, b1r):
    return pl.pallas_call(
        _l1_body,
        grid=(GRID,),
        in_specs=[
            pl.BlockSpec((BM, 128), lambda i: (i, 0)),
            pl.BlockSpec((BM, 128), lambda i: (i, 0)),
            pl.BlockSpec((BM, 128), lambda i: (i, 0)),
            pl.BlockSpec((BM, 128), lambda i: (i, 0)),
            pl.BlockSpec((128, 256), lambda i: (0, 0)),
            pl.BlockSpec((128, 256), lambda i: (0, 0)),
            pl.BlockSpec((8, 256), lambda i: (0, 0)),
        ],
        out_specs=[
            pl.BlockSpec((BM, 128), lambda i: (i, 0)),
            pl.BlockSpec((BM, 128), lambda i: (i, 0)),
        ],
        out_shape=[
            jax.ShapeDtypeStruct((NP, 128), _F32),
            jax.ShapeDtypeStruct((NP, 128), _F32),
        ],
    )(cntb, p0, p1, xp, Wl1, Wr1, b1r)


def _l2_body(cnt_ref, agg_ref, h1a_ref, h1b_ref, wl2_ref, wr2a_ref,
             wr2b_ref, b2_ref, wl3_ref, wr3_ref, b3_ref, p_ref, init_ref):
    cnt = cnt_ref[...][:, 0:1]
    cmax = jnp.maximum(cnt, 1.0)
    mean = agg_ref[...] * (1.0 / cmax)
    h = jnp.dot(mean, wl2_ref[...], preferred_element_type=_F32)
    h = h + jnp.dot(h1a_ref[...], wr2a_ref[...], preferred_element_type=_F32)
    h = h + jnp.dot(h1b_ref[...], wr2b_ref[...], preferred_element_type=_F32)
    h = jnp.maximum(h + b2_ref[...][0:1, :], 0.0)
    p_ref[...] = jnp.dot(h, wl3_ref[...], preferred_element_type=_F32)
    r = jnp.dot(h, wr3_ref[...], preferred_element_type=_F32)
    init_ref[...] = (r + b3_ref[...][0:1, :]) * cmax


def _tc_layer23(cntb, agg2, h1a, h1b, Wl2, Wr2a, Wr2b, b2r, Wl3p, Wr3p, b3r):
    return pl.pallas_call(
        _l2_body,
        grid=(GRID,),
        in_specs=[
            pl.BlockSpec((BM, 128), lambda i: (i, 0)),
            pl.BlockSpec((BM, 256), lambda i: (i, 0)),
            pl.BlockSpec((BM, 128), lambda i: (i, 0)),
            pl.BlockSpec((BM, 128), lambda i: (i, 0)),
            pl.BlockSpec((256, 256), lambda i: (0, 0)),
            pl.BlockSpec((128, 256), lambda i: (0, 0)),
            pl.BlockSpec((128, 256), lambda i: (0, 0)),
            pl.BlockSpec((8, 256), lambda i: (0, 0)),
            pl.BlockSpec((256, 128), lambda i: (0, 0)),
            pl.BlockSpec((256, 128), lambda i: (0, 0)),
            pl.BlockSpec((8, 128), lambda i: (0, 0)),
        ],
        out_specs=[
            pl.BlockSpec((BM, 128), lambda i: (i, 0)),
            pl.BlockSpec((BM, 128), lambda i: (i, 0)),
        ],
        out_shape=[
            jax.ShapeDtypeStruct((NP, 128), _F32),
            jax.ShapeDtypeStruct((NP, 128), _F32),
        ],
    )(cntb, agg2, h1a, h1b, Wl2, Wr2a, Wr2b, b2r, Wl3p, Wr3p, b3r)


def _finish_body(cnt_ref, a0_ref, a1_ref, out_ref):
    cnt = cnt_ref[...][:, 0:1]
    out_ref[...] = ((a0_ref[...] + a1_ref[...])
                    * (1.0 / jnp.maximum(cnt, 1.0)))


def _tc_finish(cntb, a0, a1):
    return pl.pallas_call(
        _finish_body,
        grid=(GRID,),
        in_specs=[
            pl.BlockSpec((BM, 128), lambda i: (i, 0)),
            pl.BlockSpec((BM, 16), lambda i: (i, 0)),
            pl.BlockSpec((BM, 16), lambda i: (i, 0)),
        ],
        out_specs=pl.BlockSpec((BM, 16), lambda i: (i, 0)),
        out_shape=jax.ShapeDtypeStruct((NP, 16), _F32),
    )(cntb, a0, a1)


def kernel(x, edge_index, Wl1, Wr1, b1, Wl2, Wr2, b2, Wl3, Wr3, b3):
    src = edge_index[0].astype(jnp.int32)
    dst = edge_index[1].astype(jnp.int32)

    xp = jnp.pad(x, ((0, NP - N_NODES), (0, 0)))

    p0, p1, c0, c1 = _seg_sum_l1(xp, src, dst)
    cnt = c0 + c1
    cntb = jnp.broadcast_to(cnt[:, None], (NP, 128))

    b1r = jnp.broadcast_to(b1[None, :], (8, 256))
    h1a, h1b = _tc_layer1(cntb, p0, p1, xp, Wl1, Wr1, b1r)

    agg2, = _seg_sum_128(h1a, h1b, src, dst)

    b2r = jnp.broadcast_to(b2[None, :], (8, 256))
    Wl3p = jnp.pad(Wl3, ((0, 0), (0, 126)))
    Wr3p = jnp.pad(Wr3, ((0, 0), (0, 126)))
    b3r = jnp.broadcast_to(jnp.pad(b3, (0, 126))[None, :], (8, 128))
    pout, iout = _tc_layer23(cntb, agg2, h1a, h1b, Wl2, Wr2[:128],
                             Wr2[128:], b2r, Wl3p, Wr3p, b3r)

    p16 = pout[:, :16]
    init16 = iout[:, :16]
    a0, a1 = _seg_mean16(p16, init16, src, dst)
    outp = _tc_finish(cntb, a0, a1)
    return outp[:N_NODES, :2]


# speedup vs baseline: 10.6421x; 1.0003x over previous
import functools

import jax
import jax.numpy as jnp
from jax import lax
from jax.experimental import pallas as pl
from jax.experimental.pallas import tpu as pltpu
from jax.experimental.pallas import tpu_sc as plsc

N_NODES = 10000
NP = 10240
E = 320000
EB = 128
NBATCH = E // EB
TILES = 16
ROWS_PT = NP // TILES
ZR = 160

_F32 = jnp.float32


def _fill_zero(ref, rows, cols):
    zero16 = jnp.zeros((16,), _F32)

    def body(r, _):
        for j in range(cols // 16):
            ref[r, pl.ds(j * 16, 16)] = zero16
        return 0
    lax.fori_loop(0, rows, body, 0)


def _fill_zero_1d(ref, n):
    zero16 = jnp.zeros((16,), _F32)

    def body(r, _):
        ref[pl.ds(r * 16, 16)] = zero16
        return 0
    lax.fori_loop(0, n // 16, body, 0)


def _memset_acc_rows(zsrc, acc, row0, zrows):
    off = 0
    while off < ROWS_PT:
        n = min(zrows, ROWS_PT - off)
        pltpu.sync_copy(zsrc.at[pl.ds(0, n)],
                        acc.at[pl.ds(row0 + off, n)])
        off += n


def _edge_loop(sid, table_h, src_h, dst_h, acc, srcv, dstv, rows_v,
               semi, semg, sems, k, k0b, nbatch, cacc=None, ones_v=None):
    nfull = nbatch // TILES
    rem = nbatch % TILES
    assert nfull % k == 0
    nsup = nfull // k
    o_s = k0b + sid * nfull + jnp.minimum(sid, rem)

    def fire_gather(b0, i):
        p = i % 2
        return pltpu.async_copy(
            table_h.at[srcv.at[pl.ds(i * EB, EB)]],
            rows_v.at[pl.ds(p * EB, EB)], semg)

    def fire_scatter(i):
        p = i % 2
        cps = [pltpu.async_copy(rows_v.at[pl.ds(p * EB, EB)],
                                acc.at[dstv.at[i]], sems, add=True)]
        if cacc is not None:
            cps.append(pltpu.async_copy(ones_v, cacc.at[dstv.at[i]], sems,
                                        add=True))
        return cps

    def do_group(b0, kk):
        e0 = b0 * EB
        cps = [pltpu.async_copy(src_h.at[pl.ds(e0, kk * EB)],
                                srcv.at[pl.ds(0, kk * EB)], semi)]
        cps += [pltpu.async_copy(dst_h.at[pl.ds(e0 + i * EB, EB)],
                                 dstv.at[i], semi)
                for i in range(kk)]
        for cp in cps:
            cp.wait()
        g = fire_gather(b0, 0)
        sc_prev = []
        for i in range(kk):
            g.wait()
            for cp in sc_prev:
                cp.wait()
            if i + 1 < kk:
                g = fire_gather(b0, i + 1)
            sc_prev = fire_scatter(i)
        for cp in sc_prev:
            cp.wait()

    def body(g, _):
        do_group(o_s + g * k, k)
        return 0

    lax.fori_loop(0, nsup, body, 0)

    @pl.when(sid < rem)
    def _():
        do_group(o_s + nfull, 1)


def _make_seg_sum_l1():
    mesh = plsc.VectorSubcoreMesh(core_axis_name="c", subcore_axis_name="s")
    out_type = [
        jax.ShapeDtypeStruct((NP, 128), _F32),
        jax.ShapeDtypeStruct((NP, 128), _F32),
        jax.ShapeDtypeStruct((NP,), _F32),
        jax.ShapeDtypeStruct((NP,), _F32),
    ]
    k = 13
    scratch = [
        pltpu.VMEM_SHARED((NP, 128), _F32),
        pltpu.VMEM((k * EB,), jnp.int32),
        pltpu.VMEM((k, EB), jnp.int32),
        pltpu.VMEM((2 * EB, 128), _F32),
        pltpu.SemaphoreType.DMA,
        pltpu.SemaphoreType.DMA,
        pltpu.SemaphoreType.DMA,
        pltpu.VMEM_SHARED((NP,), _F32),
        pltpu.VMEM((EB,), _F32),
        pltpu.VMEM((ROWS_PT,), _F32),
    ]

    @functools.partial(pl.kernel, mesh=mesh, out_type=out_type,
                       scratch_types=scratch)
    def seg_sum_l1(x_h, src_h, dst_h, p0_h, p1_h, c0_h, c1_h,
                   acc, srcv, dstv, rows_v, semi, semg, sems,
                   cacc, ones_v, zrow):
        cid = lax.axis_index("c")
        sid = lax.axis_index("s")
        row0 = sid * ROWS_PT

        _fill_zero(rows_v, 2 * EB, 128)
        _memset_acc_rows(rows_v, acc, row0, 2 * EB)
        _fill_zero_1d(zrow, ROWS_PT)
        pltpu.sync_copy(zrow, cacc.at[pl.ds(row0, ROWS_PT)])

        one16 = jnp.ones((16,), _F32)

        def ones_body(j, _):
            ones_v[pl.ds(j * 16, 16)] = one16
            return 0
        lax.fori_loop(0, EB // 16, ones_body, 0)
        plsc.subcore_barrier()

        half = NBATCH // 2

        @pl.when(cid == 0)
        def _():
            _edge_loop(sid, x_h, src_h, dst_h, acc, srcv, dstv, rows_v,
                       semi, semg, sems, k, 0, half, cacc, ones_v)

        @pl.when(cid == 1)
        def _():
            _edge_loop(sid, x_h, src_h, dst_h, acc, srcv, dstv, rows_v,
                       semi, semg, sems, k, half, half, cacc, ones_v)

        plsc.subcore_barrier()

        @pl.when(cid == 0)
        def _():
            pltpu.sync_copy(acc.at[pl.ds(row0, ROWS_PT)],
                            p0_h.at[pl.ds(row0, ROWS_PT)])
            pltpu.sync_copy(cacc.at[pl.ds(row0, ROWS_PT)],
                            c0_h.at[pl.ds(row0, ROWS_PT)])

        @pl.when(cid == 1)
        def _():
            pltpu.sync_copy(acc.at[pl.ds(row0, ROWS_PT)],
                            p1_h.at[pl.ds(row0, ROWS_PT)])
            pltpu.sync_copy(cacc.at[pl.ds(row0, ROWS_PT)],
                            c1_h.at[pl.ds(row0, ROWS_PT)])

    return seg_sum_l1


def _make_seg_sum_fsplit():
    mesh = plsc.VectorSubcoreMesh(core_axis_name="c", subcore_axis_name="s")
    out_type = [jax.ShapeDtypeStruct((NP, 256), _F32)]
    k = 12
    scratch = [
        pltpu.VMEM_SHARED((NP, 128), _F32),
        pltpu.VMEM((k * EB,), jnp.int32),
        pltpu.VMEM((k, EB), jnp.int32),
        pltpu.VMEM((2 * EB, 128), _F32),
        pltpu.SemaphoreType.DMA,
        pltpu.SemaphoreType.DMA,
        pltpu.SemaphoreType.DMA,
    ]

    @functools.partial(pl.kernel, mesh=mesh, out_type=out_type,
                       scratch_types=scratch)
    def seg_sum_fs(t0_h, t1_h, src_h, dst_h, out_h,
                   acc, srcv, dstv, rows_v, semi, semg, sems):
        cid = lax.axis_index("c")
        sid = lax.axis_index("s")
        row0 = sid * ROWS_PT

        _fill_zero(rows_v, 2 * EB, 128)
        _memset_acc_rows(rows_v, acc, row0, 2 * EB)
        plsc.subcore_barrier()

        @pl.when(cid == 0)
        def _():
            _edge_loop(sid, t0_h, src_h, dst_h, acc, srcv, dstv, rows_v,
                       semi, semg, sems, k, 0, NBATCH)

        @pl.when(cid == 1)
        def _():
            _edge_loop(sid, t1_h, src_h, dst_h, acc, srcv, dstv, rows_v,
                       semi, semg, sems, k, 0, NBATCH)

        plsc.subcore_barrier()

        @pl.when(cid == 0)
        def _():
            pltpu.sync_copy(acc.at[pl.ds(row0, ROWS_PT)],
                            out_h.at[pl.ds(row0, ROWS_PT), pl.ds(0, 128)])

        @pl.when(cid == 1)
        def _():
            pltpu.sync_copy(acc.at[pl.ds(row0, ROWS_PT)],
                            out_h.at[pl.ds(row0, ROWS_PT), pl.ds(128, 128)])

    return seg_sum_fs


_seg_sum_l1 = _make_seg_sum_l1()
_seg_sum_128 = _make_seg_sum_fsplit()


def _make_seg_mean16():
    mesh = plsc.VectorSubcoreMesh(core_axis_name="c", subcore_axis_name="s")
    out_type = [
        jax.ShapeDtypeStruct((NP, 16), _F32),
        jax.ShapeDtypeStruct((NP, 16), _F32),
    ]
    k = 6
    scratch = [
        pltpu.VMEM_SHARED((NP, 16), _F32),
        pltpu.VMEM((k * EB,), jnp.int32),
        pltpu.VMEM((k, EB), jnp.int32),
        pltpu.VMEM((2 * EB, 16), _F32),
        pltpu.SemaphoreType.DMA,
        pltpu.SemaphoreType.DMA,
        pltpu.SemaphoreType.DMA,
    ]

    @functools.partial(
        pl.kernel, mesh=mesh, out_type=out_type, scratch_types=scratch,
        compiler_params=pltpu.CompilerParams(use_tc_tiling_on_sc=False))
    def seg_mean16(p_h, init_h, src_h, dst_h, o0_h, o1_h,
                   acc, srcv, dstv, rows_v, semi, semg, sems):
        cid = lax.axis_index("c")
        sid = lax.axis_index("s")
        row0 = sid * ROWS_PT
        half = NBATCH // 2

        @pl.when(cid == 0)
        def _():
            pltpu.sync_copy(init_h.at[pl.ds(row0, ROWS_PT)],
                            acc.at[pl.ds(row0, ROWS_PT)])
            plsc.subcore_barrier()
            _edge_loop(sid, p_h, src_h, dst_h, acc, srcv, dstv, rows_v,
                       semi, semg, sems, k, 0, half)
            plsc.subcore_barrier()
            pltpu.sync_copy(acc.at[pl.ds(row0, ROWS_PT)],
                            o0_h.at[pl.ds(row0, ROWS_PT)])

        @pl.when(cid == 1)
        def _():
            _fill_zero(rows_v, 2 * EB, 16)
            _memset_acc_rows(rows_v, acc, row0, 2 * EB)
            plsc.subcore_barrier()
            _edge_loop(sid, p_h, src_h, dst_h, acc, srcv, dstv, rows_v,
                       semi, semg, sems, k, half, half)
            plsc.subcore_barrier()
            pltpu.sync_copy(acc.at[pl.ds(row0, ROWS_PT)],
                            o1_h.at[pl.ds(row0, ROWS_PT)])

    return seg_mean16


_seg_mean16 = _make_seg_mean16()

BM = 256
GRID = NP // BM


def _l1_body(cnt_ref, p0_ref, p1_ref, x_ref, wl_ref, wr_ref, b_ref,
             h1a_ref, h1b_ref):
    cnt = cnt_ref[...][:, 0:1]
    agg = p0_ref[...] + p1_ref[...]
    mean = agg * (1.0 / jnp.maximum(cnt, 1.0))
    h = jnp.dot(mean, wl_ref[...], preferred_element_type=_F32)
    h = h + jnp.dot(x_ref[...], wr_ref[...], preferred_element_type=_F32)
    h = jnp.maximum(h + b_ref[...][0:1, :], 0.0)
    h1a_ref[...] = h[:, :128]
    h1b_ref[...] = h[:, 128:]


def _tc_layer1(cntb, p0, p1, xp, Wl1, Wr1, b1r):
    return pl.pallas_call(
        _l1_body,
        grid=(GRID,),
        in_specs=[
            pl.BlockSpec((BM, 128), lambda i: (i, 0)),
            pl.BlockSpec((BM, 128), lambda i: (i, 0)),
            pl.BlockSpec((BM, 128), lambda i: (i, 0)),
            pl.BlockSpec((BM, 128), lambda i: (i, 0)),
            pl.BlockSpec((128, 256), lambda i: (0, 0)),
            pl.BlockSpec((128, 256), lambda i: (0, 0)),
            pl.BlockSpec((8, 256), lambda i: (0, 0)),
        ],
        out_specs=[
            pl.BlockSpec((BM, 128), lambda i: (i, 0)),
            pl.BlockSpec((BM, 128), lambda i: (i, 0)),
        ],
        out_shape=[
            jax.ShapeDtypeStruct((NP, 128), _F32),
            jax.ShapeDtypeStruct((NP, 128), _F32),
        ],
    )(cntb, p0, p1, xp, Wl1, Wr1, b1r)


def _l2_body(cnt_ref, agg_ref, h1a_ref, h1b_ref, wl2_ref, wr2a_ref,
             wr2b_ref, b2_ref, wl3_ref, wr3_ref, b3_ref, p_ref, init_ref):
    cnt = cnt_ref[...][:, 0:1]
    cmax = jnp.maximum(cnt, 1.0)
    mean = agg_ref[...] * (1.0 / cmax)
    h = jnp.dot(mean, wl2_ref[...], preferred_element_type=_F32)
    h = h + jnp.dot(h1a_ref[...], wr2a_ref[...], preferred_element_type=_F32)
    h = h + jnp.dot(h1b_ref[...], wr2b_ref[...], preferred_element_type=_F32)
    h = jnp.maximum(h + b2_ref[...][0:1, :], 0.0)
    p_ref[...] = jnp.dot(h, wl3_ref[...], preferred_element_type=_F32)
    r = jnp.dot(h, wr3_ref[...], preferred_element_type=_F32)
    init_ref[...] = (r + b3_ref[...][0:1, :]) * cmax


def _tc_layer23(cntb, agg2, h1a, h1b, Wl2, Wr2a, Wr2b, b2r, Wl3p, Wr3p, b3r):
    return pl.pallas_call(
        _l2_body,
        grid=(GRID,),
        in_specs=[
            pl.BlockSpec((BM, 128), lambda i: (i, 0)),
            pl.BlockSpec((BM, 256), lambda i: (i, 0)),
            pl.BlockSpec((BM, 128), lambda i: (i, 0)),
            pl.BlockSpec((BM, 128), lambda i: (i, 0)),
            pl.BlockSpec((256, 256), lambda i: (0, 0)),
            pl.BlockSpec((128, 256), lambda i: (0, 0)),
            pl.BlockSpec((128, 256), lambda i: (0, 0)),
            pl.BlockSpec((8, 256), lambda i: (0, 0)),
            pl.BlockSpec((256, 128), lambda i: (0, 0)),
            pl.BlockSpec((256, 128), lambda i: (0, 0)),
            pl.BlockSpec((8, 128), lambda i: (0, 0)),
        ],
        out_specs=[
            pl.BlockSpec((BM, 128), lambda i: (i, 0)),
            pl.BlockSpec((BM, 128), lambda i: (i, 0)),
        ],
        out_shape=[
            jax.ShapeDtypeStruct((NP, 128), _F32),
            jax.ShapeDtypeStruct((NP, 128), _F32),
        ],
    )(cntb, agg2, h1a, h1b, Wl2, Wr2a, Wr2b, b2r, Wl3p, Wr3p, b3r)


def _finish_body(cnt_ref, a0_ref, a1_ref, out_ref):
    cnt = cnt_ref[...][:, 0:1]
    out_ref[...] = ((a0_ref[...] + a1_ref[...])
                    * (1.0 / jnp.maximum(cnt, 1.0)))


def _tc_finish(cntb, a0, a1):
    return pl.pallas_call(
        _finish_body,
        grid=(GRID,),
        in_specs=[
            pl.BlockSpec((BM, 128), lambda i: (i, 0)),
            pl.BlockSpec((BM, 16), lambda i: (i, 0)),
            pl.BlockSpec((BM, 16), lambda i: (i, 0)),
        ],
        out_specs=pl.BlockSpec((BM, 16), lambda i: (i, 0)),
        out_shape=jax.ShapeDtypeStruct((NP, 16), _F32),
    )(cntb, a0, a1)


def kernel(x, edge_index, Wl1, Wr1, b1, Wl2, Wr2, b2, Wl3, Wr3, b3):
    src = edge_index[0].astype(jnp.int32)
    dst = edge_index[1].astype(jnp.int32)

    xp = jnp.pad(x, ((0, NP - N_NODES), (0, 0)))

    p0, p1, c0, c1 = _seg_sum_l1(xp, src, dst)
    cnt = c0 + c1
    cntb = jnp.broadcast_to(cnt[:, None], (NP, 128))

    b1r = jnp.broadcast_to(b1[None, :], (8, 256))
    h1a, h1b = _tc_layer1(cntb, p0, p1, xp, Wl1, Wr1, b1r)

    agg2, = _seg_sum_128(h1a, h1b, src, dst)

    b2r = jnp.broadcast_to(b2[None, :], (8, 256))
    Wl3p = jnp.pad(Wl3, ((0, 0), (0, 126)))
    Wr3p = jnp.pad(Wr3, ((0, 0), (0, 126)))
    b3r = jnp.broadcast_to(jnp.pad(b3, (0, 126))[None, :], (8, 128))
    pout, iout = _tc_layer23(cntb, agg2, h1a, h1b, Wl2, Wr2[:128],
                             Wr2[128:], b2r, Wl3p, Wr3p, b3r)

    p16 = pout[:, :16]
    init16 = iout[:, :16]
    a0, a1 = _seg_mean16(p16, init16, src, dst)
    outp = _tc_finish(cntb, a0, a1)
    return outp[:N_NODES, :2]


# speedup vs baseline: 10.6811x; 1.0037x over previous
import functools

import jax
import jax.numpy as jnp
from jax import lax
from jax.experimental import pallas as pl
from jax.experimental.pallas import tpu as pltpu
from jax.experimental.pallas import tpu_sc as plsc

N_NODES = 10000
NP = 10240
E = 320000
EB = 128
NBATCH = E // EB
TILES = 16
ROWS_PT = NP // TILES
ZR = 160

_F32 = jnp.float32


def _fill_zero(ref, rows, cols):
    zero16 = jnp.zeros((16,), _F32)

    def body(r, _):
        for j in range(cols // 16):
            ref[r, pl.ds(j * 16, 16)] = zero16
        return 0
    lax.fori_loop(0, rows, body, 0)


def _fill_zero_1d(ref, n):
    zero16 = jnp.zeros((16,), _F32)

    def body(r, _):
        ref[pl.ds(r * 16, 16)] = zero16
        return 0
    lax.fori_loop(0, n // 16, body, 0)


def _memset_acc_rows(zsrc, acc, row0, zrows):
    off = 0
    while off < ROWS_PT:
        n = min(zrows, ROWS_PT - off)
        pltpu.sync_copy(zsrc.at[pl.ds(0, n)],
                        acc.at[pl.ds(row0 + off, n)])
        off += n


def _edge_loop(sid, table_h, src_h, dst_h, acc, srcv, dstv, rows_v,
               semi, semg, sems, k, k0b, nbatch, cacc=None, ones_v=None):
    nfull = nbatch // TILES
    rem = nbatch % TILES
    assert nfull % k == 0
    nsup = nfull // k
    o_s = k0b + sid * nfull + jnp.minimum(sid, rem)

    def fire_gather(b0, i):
        p = i % 2
        return pltpu.async_copy(
            table_h.at[srcv.at[pl.ds(i * EB, EB)]],
            rows_v.at[pl.ds(p * EB, EB)], semg)

    def fire_scatter(i):
        p = i % 2
        cps = [pltpu.async_copy(rows_v.at[pl.ds(p * EB, EB)],
                                acc.at[dstv.at[i]], sems, add=True)]
        if cacc is not None:
            cps.append(pltpu.async_copy(ones_v, cacc.at[dstv.at[i]], sems,
                                        add=True))
        return cps

    def do_group(b0, kk):
        e0 = b0 * EB
        cps = [pltpu.async_copy(src_h.at[pl.ds(e0, kk * EB)],
                                srcv.at[pl.ds(0, kk * EB)], semi)]
        cps += [pltpu.async_copy(dst_h.at[pl.ds(e0 + i * EB, EB)],
                                 dstv.at[i], semi)
                for i in range(kk)]
        for cp in cps:
            cp.wait()
        g = fire_gather(b0, 0)
        sc_prev = []
        for i in range(kk):
            g.wait()
            for cp in sc_prev:
                cp.wait()
            if i + 1 < kk:
                g = fire_gather(b0, i + 1)
            sc_prev = fire_scatter(i)
        for cp in sc_prev:
            cp.wait()

    def body(g, _):
        do_group(o_s + g * k, k)
        return 0

    lax.fori_loop(0, nsup, body, 0)

    @pl.when(sid < rem)
    def _():
        do_group(o_s + nfull, 1)


def _make_seg_sum_l1():
    mesh = plsc.VectorSubcoreMesh(core_axis_name="c", subcore_axis_name="s")
    out_type = [
        jax.ShapeDtypeStruct((NP, 128), _F32),
        jax.ShapeDtypeStruct((NP, 128), _F32),
        jax.ShapeDtypeStruct((NP,), _F32),
        jax.ShapeDtypeStruct((NP,), _F32),
    ]
    k = 13
    scratch = [
        pltpu.VMEM_SHARED((NP, 128), _F32),
        pltpu.VMEM((k * EB,), jnp.int32),
        pltpu.VMEM((k, EB), jnp.int32),
        pltpu.VMEM((2 * EB, 128), _F32),
        pltpu.SemaphoreType.DMA,
        pltpu.SemaphoreType.DMA,
        pltpu.SemaphoreType.DMA,
        pltpu.VMEM_SHARED((NP,), _F32),
        pltpu.VMEM((EB,), _F32),
        pltpu.VMEM((ROWS_PT,), _F32),
    ]

    @functools.partial(pl.kernel, mesh=mesh, out_type=out_type,
                       scratch_types=scratch)
    def seg_sum_l1(x_h, src_h, dst_h, p0_h, p1_h, c0_h, c1_h,
                   acc, srcv, dstv, rows_v, semi, semg, sems,
                   cacc, ones_v, zrow):
        cid = lax.axis_index("c")
        sid = lax.axis_index("s")
        row0 = sid * ROWS_PT

        _fill_zero(rows_v, 2 * EB, 128)
        _memset_acc_rows(rows_v, acc, row0, 2 * EB)
        _fill_zero_1d(zrow, ROWS_PT)
        pltpu.sync_copy(zrow, cacc.at[pl.ds(row0, ROWS_PT)])

        one16 = jnp.ones((16,), _F32)

        def ones_body(j, _):
            ones_v[pl.ds(j * 16, 16)] = one16
            return 0
        lax.fori_loop(0, EB // 16, ones_body, 0)
        plsc.subcore_barrier()

        half = NBATCH // 2

        @pl.when(cid == 0)
        def _():
            _edge_loop(sid, x_h, src_h, dst_h, acc, srcv, dstv, rows_v,
                       semi, semg, sems, k, 0, half, cacc, ones_v)

        @pl.when(cid == 1)
        def _():
            _edge_loop(sid, x_h, src_h, dst_h, acc, srcv, dstv, rows_v,
                       semi, semg, sems, k, half, half, cacc, ones_v)

        plsc.subcore_barrier()

        @pl.when(cid == 0)
        def _():
            pltpu.sync_copy(acc.at[pl.ds(row0, ROWS_PT)],
                            p0_h.at[pl.ds(row0, ROWS_PT)])
            pltpu.sync_copy(cacc.at[pl.ds(row0, ROWS_PT)],
                            c0_h.at[pl.ds(row0, ROWS_PT)])

        @pl.when(cid == 1)
        def _():
            pltpu.sync_copy(acc.at[pl.ds(row0, ROWS_PT)],
                            p1_h.at[pl.ds(row0, ROWS_PT)])
            pltpu.sync_copy(cacc.at[pl.ds(row0, ROWS_PT)],
                            c1_h.at[pl.ds(row0, ROWS_PT)])

    return seg_sum_l1


def _make_seg_sum_fsplit():
    mesh = plsc.VectorSubcoreMesh(core_axis_name="c", subcore_axis_name="s")
    out_type = [jax.ShapeDtypeStruct((NP, 256), _F32)]
    k = 12
    scratch = [
        pltpu.VMEM_SHARED((NP, 128), _F32),
        pltpu.VMEM((k * EB,), jnp.int32),
        pltpu.VMEM((k, EB), jnp.int32),
        pltpu.VMEM((2 * EB, 128), _F32),
        pltpu.SemaphoreType.DMA,
        pltpu.SemaphoreType.DMA,
        pltpu.SemaphoreType.DMA,
    ]

    @functools.partial(pl.kernel, mesh=mesh, out_type=out_type,
                       scratch_types=scratch)
    def seg_sum_fs(t0_h, t1_h, src_h, dst_h, out_h,
                   acc, srcv, dstv, rows_v, semi, semg, sems):
        cid = lax.axis_index("c")
        sid = lax.axis_index("s")
        row0 = sid * ROWS_PT

        _fill_zero(rows_v, 2 * EB, 128)
        _memset_acc_rows(rows_v, acc, row0, 2 * EB)
        plsc.subcore_barrier()

        @pl.when(cid == 0)
        def _():
            _edge_loop(sid, t0_h, src_h, dst_h, acc, srcv, dstv, rows_v,
                       semi, semg, sems, k, 0, NBATCH)

        @pl.when(cid == 1)
        def _():
            _edge_loop(sid, t1_h, src_h, dst_h, acc, srcv, dstv, rows_v,
                       semi, semg, sems, k, 0, NBATCH)

        plsc.subcore_barrier()

        @pl.when(cid == 0)
        def _():
            pltpu.sync_copy(acc.at[pl.ds(row0, ROWS_PT)],
                            out_h.at[pl.ds(row0, ROWS_PT), pl.ds(0, 128)])

        @pl.when(cid == 1)
        def _():
            pltpu.sync_copy(acc.at[pl.ds(row0, ROWS_PT)],
                            out_h.at[pl.ds(row0, ROWS_PT), pl.ds(128, 128)])

    return seg_sum_fs


_seg_sum_l1 = _make_seg_sum_l1()
_seg_sum_128 = _make_seg_sum_fsplit()


def _make_seg_mean16():
    mesh = plsc.VectorSubcoreMesh(core_axis_name="c", subcore_axis_name="s")
    out_type = [
        jax.ShapeDtypeStruct((NP, 16), _F32),
        jax.ShapeDtypeStruct((NP, 16), _F32),
    ]
    k = 13
    scratch = [
        pltpu.VMEM_SHARED((NP, 16), _F32),
        pltpu.VMEM((k * EB,), jnp.int32),
        pltpu.VMEM((k, EB), jnp.int32),
        pltpu.VMEM((2 * EB, 16), _F32),
        pltpu.SemaphoreType.DMA,
        pltpu.SemaphoreType.DMA,
        pltpu.SemaphoreType.DMA,
    ]

    @functools.partial(
        pl.kernel, mesh=mesh, out_type=out_type, scratch_types=scratch,
        compiler_params=pltpu.CompilerParams(use_tc_tiling_on_sc=False))
    def seg_mean16(p_h, init_h, src_h, dst_h, o0_h, o1_h,
                   acc, srcv, dstv, rows_v, semi, semg, sems):
        cid = lax.axis_index("c")
        sid = lax.axis_index("s")
        row0 = sid * ROWS_PT
        half = NBATCH // 2

        @pl.when(cid == 0)
        def _():
            pltpu.sync_copy(init_h.at[pl.ds(row0, ROWS_PT)],
                            acc.at[pl.ds(row0, ROWS_PT)])
            plsc.subcore_barrier()
            _edge_loop(sid, p_h, src_h, dst_h, acc, srcv, dstv, rows_v,
                       semi, semg, sems, k, 0, half)
            plsc.subcore_barrier()
            pltpu.sync_copy(acc.at[pl.ds(row0, ROWS_PT)],
                            o0_h.at[pl.ds(row0, ROWS_PT)])

        @pl.when(cid == 1)
        def _():
            _fill_zero(rows_v, 2 * EB, 16)
            _memset_acc_rows(rows_v, acc, row0, 2 * EB)
            plsc.subcore_barrier()
            _edge_loop(sid, p_h, src_h, dst_h, acc, srcv, dstv, rows_v,
                       semi, semg, sems, k, half, half)
            plsc.subcore_barrier()
            pltpu.sync_copy(acc.at[pl.ds(row0, ROWS_PT)],
                            o1_h.at[pl.ds(row0, ROWS_PT)])

    return seg_mean16


_seg_mean16 = _make_seg_mean16()

BM = 256
GRID = NP // BM


def _l1_body(cnt_ref, p0_ref, p1_ref, x_ref, wl_ref, wr_ref, b_ref,
             h1a_ref, h1b_ref):
    cnt = cnt_ref[...][:, 0:1]
    agg = p0_ref[...] + p1_ref[...]
    mean = agg * (1.0 / jnp.maximum(cnt, 1.0))
    h = jnp.dot(mean, wl_ref[...], preferred_element_type=_F32)
    h = h + jnp.dot(x_ref[...], wr_ref[...], preferred_element_type=_F32)
    h = jnp.maximum(h + b_ref[...][0:1, :], 0.0)
    h1a_ref[...] = h[:, :128]
    h1b_ref[...] = h[:, 128:]


def _tc_layer1(cntb, p0, p1, xp, Wl1, Wr1, b1r):
    return pl.pallas_call(
        _l1_body,
        grid=(GRID,),
        in_specs=[
            pl.BlockSpec((BM, 128), lambda i: (i, 0)),
            pl.BlockSpec((BM, 128), lambda i: (i, 0)),
            pl.BlockSpec((BM, 128), lambda i: (i, 0)),
            pl.BlockSpec((BM, 128), lambda i: (i, 0)),
            pl.BlockSpec((128, 256), lambda i: (0, 0)),
            pl.BlockSpec((128, 256), lambda i: (0, 0)),
            pl.BlockSpec((8, 256), lambda i: (0, 0)),
        ],
        out_specs=[
            pl.BlockSpec((BM, 128), lambda i: (i, 0)),
            pl.BlockSpec((BM, 128), lambda i: (i, 0)),
        ],
        out_shape=[
            jax.ShapeDtypeStruct((NP, 128), _F32),
            jax.ShapeDtypeStruct((NP, 128), _F32),
        ],
    )(cntb, p0, p1, xp, Wl1, Wr1, b1r)


def _l2_body(cnt_ref, agg_ref, h1a_ref, h1b_ref, wl2_ref, wr2a_ref,
             wr2b_ref, b2_ref, wl3_ref, wr3_ref, b3_ref, p_ref, init_ref):
    cnt = cnt_ref[...][:, 0:1]
    cmax = jnp.maximum(cnt, 1.0)
    mean = agg_ref[...] * (1.0 / cmax)
    h = jnp.dot(mean, wl2_ref[...], preferred_element_type=_F32)
    h = h + jnp.dot(h1a_ref[...], wr2a_ref[...], preferred_element_type=_F32)
    h = h + jnp.dot(h1b_ref[...], wr2b_ref[...], preferred_element_type=_F32)
    h = jnp.maximum(h + b2_ref[...][0:1, :], 0.0)
    p_ref[...] = jnp.dot(h, wl3_ref[...], preferred_element_type=_F32)
    r = jnp.dot(h, wr3_ref[...], preferred_element_type=_F32)
    init_ref[...] = (r + b3_ref[...][0:1, :]) * cmax


def _tc_layer23(cntb, agg2, h1a, h1b, Wl2, Wr2a, Wr2b, b2r, Wl3p, Wr3p, b3r):
    return pl.pallas_call(
        _l2_body,
        grid=(GRID,),
        in_specs=[
            pl.BlockSpec((BM, 128), lambda i: (i, 0)),
            pl.BlockSpec((BM, 256), lambda i: (i, 0)),
            pl.BlockSpec((BM, 128), lambda i: (i, 0)),
            pl.BlockSpec((BM, 128), lambda i: (i, 0)),
            pl.BlockSpec((256, 256), lambda i: (0, 0)),
            pl.BlockSpec((128, 256), lambda i: (0, 0)),
            pl.BlockSpec((128, 256), lambda i: (0, 0)),
            pl.BlockSpec((8, 256), lambda i: (0, 0)),
            pl.BlockSpec((256, 128), lambda i: (0, 0)),
            pl.BlockSpec((256, 128), lambda i: (0, 0)),
            pl.BlockSpec((8, 128), lambda i: (0, 0)),
        ],
        out_specs=[
            pl.BlockSpec((BM, 128), lambda i: (i, 0)),
            pl.BlockSpec((BM, 128), lambda i: (i, 0)),
        ],
        out_shape=[
            jax.ShapeDtypeStruct((NP, 128), _F32),
            jax.ShapeDtypeStruct((NP, 128), _F32),
        ],
    )(cntb, agg2, h1a, h1b, Wl2, Wr2a, Wr2b, b2r, Wl3p, Wr3p, b3r)


def _finish_body(cnt_ref, a0_ref, a1_ref, out_ref):
    cnt = cnt_ref[...][:, 0:1]
    out_ref[...] = ((a0_ref[...] + a1_ref[...])
                    * (1.0 / jnp.maximum(cnt, 1.0)))


def _tc_finish(cntb, a0, a1):
    return pl.pallas_call(
        _finish_body,
        grid=(GRID,),
        in_specs=[
            pl.BlockSpec((BM, 128), lambda i: (i, 0)),
            pl.BlockSpec((BM, 16), lambda i: (i, 0)),
            pl.BlockSpec((BM, 16), lambda i: (i, 0)),
        ],
        out_specs=pl.BlockSpec((BM, 16), lambda i: (i, 0)),
        out_shape=jax.ShapeDtypeStruct((NP, 16), _F32),
    )(cntb, a0, a1)


def kernel(x, edge_index, Wl1, Wr1, b1, Wl2, Wr2, b2, Wl3, Wr3, b3):
    src = edge_index[0].astype(jnp.int32)
    dst = edge_index[1].astype(jnp.int32)

    xp = jnp.pad(x, ((0, NP - N_NODES), (0, 0)))

    p0, p1, c0, c1 = _seg_sum_l1(xp, src, dst)
    cnt = c0 + c1
    cntb = jnp.broadcast_to(cnt[:, None], (NP, 128))

    b1r = jnp.broadcast_to(b1[None, :], (8, 256))
    h1a, h1b = _tc_layer1(cntb, p0, p1, xp, Wl1, Wr1, b1r)

    agg2, = _seg_sum_128(h1a, h1b, src, dst)

    b2r = jnp.broadcast_to(b2[None, :], (8, 256))
    Wl3p = jnp.pad(Wl3, ((0, 0), (0, 126)))
    Wr3p = jnp.pad(Wr3, ((0, 0), (0, 126)))
    b3r = jnp.broadcast_to(jnp.pad(b3, (0, 126))[None, :], (8, 128))
    pout, iout = _tc_layer23(cntb, agg2, h1a, h1b, Wl2, Wr2[:128],
                             Wr2[128:], b2r, Wl3p, Wr3p, b3r)

    p16 = pout[:, :16]
    init16 = iout[:, :16]
    a0, a1 = _seg_mean16(p16, init16, src, dst)
    outp = _tc_finish(cntb, a0, a1)
    return outp[:N_NODES, :2]
